# baseline (speedup 1.0000x reference)
import functools

import jax
import jax.numpy as jnp
from jax import lax
from jax.experimental import pallas as pl
from jax.experimental.pallas import tpu as pltpu
from jax.experimental.pallas import tpu_sc as plsc

C = 16
W = 2048
H = 2048
N = 1048576
NW = 32
PER_W = N // NW
CHUNK = 512
NCHUNK = PER_W // CHUNK
L = 16
XROWS_PER_TILE = W // 16
HHALF = 1024


def _sampler_body(tex_hbm, u_hbm, v_hbm, out_hbm,
                  tab_hbm,
                  chb, rows_v,
                  u_v, v_v,
                  i00, i10, i01, i11,
                  w00, w10, w01, w11,
                  f00, f10, f01, f11,
                  out_v, sem):
    cc = lax.axis_index("c")
    sid = lax.axis_index("s")
    coff = cc * (W * H)

    io1 = lax.iota(jnp.int32, L)

    def xrow_body(r, _):
        x = sid * XROWS_PER_TILE + r
        cps = [pltpu.async_copy(tex_hbm.at[c16, x, :], chb.at[c16], sem)
               for c16 in range(C)]
        for cp in cps:
            cp.wait()
        for h in range(2):
            def grp_body(g, _):
                rowi = io1 + g * L
                for c16 in range(C):
                    v = chb[c16, pl.ds(h * HHALF + g * L, L)]
                    plsc.store_scatter(
                        rows_v, [rowi, jnp.full((L,), c16, jnp.int32)], v)
                return 0
            lax.fori_loop(0, HHALF // L, grp_body, 0)
            base = coff + x * H + h * HHALF
            pltpu.sync_copy(rows_v, tab_hbm.at[pl.ds(base, HHALF)])
        return 0

    lax.fori_loop(0, XROWS_PER_TILE, xrow_body, 0)
    plsc.subcore_barrier()

    wid = sid * 2 + cc
    qbase = wid * PER_W

    def chunk_body(ci, _):
        off = qbase + ci * CHUNK
        pltpu.sync_copy(u_hbm.at[pl.ds(off, CHUNK)], u_v)
        pltpu.sync_copy(v_hbm.at[pl.ds(off, CHUNK)], v_v)

        def grp_body(gi, _):
            s = pl.ds(gi * L, L)
            u = u_v[s]
            v = v_v[s]
            x = u * jnp.float32(W - 1)
            y = v * jnp.float32(H - 1)
            x0 = jnp.minimum(x.astype(jnp.int32), W - 1)
            y0 = jnp.minimum(y.astype(jnp.int32), H - 1)
            x1 = jnp.minimum(x0 + 1, W - 1)
            y1 = jnp.minimum(y0 + 1, H - 1)
            wx = x - x0.astype(jnp.float32)
            wy = y - y0.astype(jnp.float32)
            omx = 1.0 - wx
            omy = 1.0 - wy
            xr0 = x0 * H + coff
            xr1 = x1 * H + coff
            i00[s] = xr0 + y0
            i10[s] = xr1 + y0
            i01[s] = xr0 + y1
            i11[s] = xr1 + y1
            w00[s] = omx * omy
            w10[s] = wx * omy
            w01[s] = omx * wy
            w11[s] = wx * wy
            return 0

        lax.fori_loop(0, CHUNK // L, grp_body, 0, unroll=2)

        cp0 = pltpu.async_copy(tab_hbm.at[i00], f00, sem)
        cp1 = pltpu.async_copy(tab_hbm.at[i10], f10, sem)
        cp2 = pltpu.async_copy(tab_hbm.at[i01], f01, sem)
        cp3 = pltpu.async_copy(tab_hbm.at[i11], f11, sem)
        cp0.wait()
        cp1.wait()
        cp2.wait()
        cp3.wait()

        def comb_body(gi, _):
            s = pl.ds(gi * L, L)
            w00v = w00[s]
            w10v = w10[s]
            w01v = w01[s]
            w11v = w11[s]
            for k in range(L):
                j = gi * L + k
                acc = (f00[j, :] * w00v[k] + f10[j, :] * w10v[k]
                       + f01[j, :] * w01v[k] + f11[j, :] * w11v[k])
                out_v[j, :] = acc
            return 0

        lax.fori_loop(0, CHUNK // L, comb_body, 0)

        pltpu.sync_copy(out_v, out_hbm.at[pl.ds(off, CHUNK)])
        return 0

    lax.fori_loop(0, NCHUNK, chunk_body, 0)


def kernel(input, param):
    u = param[:, 0]
    v = param[:, 1]

    mesh = plsc.VectorSubcoreMesh(core_axis_name="c", subcore_axis_name="s")
    f = pl.kernel(
        _sampler_body,
        out_type=jax.ShapeDtypeStruct((N, C), jnp.float32),
        mesh=mesh,
        compiler_params=pltpu.CompilerParams(use_tc_tiling_on_sc=False,
                                              needs_layout_passes=False),
        scratch_types=[
            pltpu.HBM((2 * W * H, C), jnp.float32),
            pltpu.VMEM((C, H), jnp.float32),
            pltpu.VMEM((HHALF, C), jnp.float32),
            pltpu.VMEM((CHUNK,), jnp.float32),
            pltpu.VMEM((CHUNK,), jnp.float32),
            pltpu.VMEM((CHUNK,), jnp.int32),
            pltpu.VMEM((CHUNK,), jnp.int32),
            pltpu.VMEM((CHUNK,), jnp.int32),
            pltpu.VMEM((CHUNK,), jnp.int32),
            pltpu.VMEM((CHUNK,), jnp.float32),
            pltpu.VMEM((CHUNK,), jnp.float32),
            pltpu.VMEM((CHUNK,), jnp.float32),
            pltpu.VMEM((CHUNK,), jnp.float32),
            pltpu.VMEM((CHUNK, C), jnp.float32),
            pltpu.VMEM((CHUNK, C), jnp.float32),
            pltpu.VMEM((CHUNK, C), jnp.float32),
            pltpu.VMEM((CHUNK, C), jnp.float32),
            pltpu.VMEM((CHUNK, C), jnp.float32),
            pltpu.SemaphoreType.DMA,
        ],
    )
    return f(input, u, v)


# speedup vs baseline: 1.2494x; 1.2494x over previous
import functools

import jax
import jax.numpy as jnp
from jax import lax
from jax.experimental import pallas as pl
from jax.experimental.pallas import tpu as pltpu
from jax.experimental.pallas import tpu_sc as plsc

C = 16
W = 2048
H = 2048
N = 1048576
NW = 32
PER_W = N // NW
CHUNK = 256
NCHUNK = PER_W // CHUNK
L = 16
XPT = W // 16
HH = 1024


def _interleave(chb, rows, io1):
    def grp(g, _):
        rowi = io1 + g * L
        for c16 in range(C):
            v = chb[c16, pl.ds(g * L, L)]
            plsc.store_scatter(
                rows, [rowi, jnp.full((L,), c16, jnp.int32)], v)
        return 0
    lax.fori_loop(0, HH // L, grp, 0)


def _indices_weights(u_v, v_v, qo, i00, i10, i01, i11, w00, w10, w01, w11):
    def grp(gi, _):
        s = pl.ds(gi * L, L)
        u = u_v[pl.ds(qo + gi * L, L)]
        v = v_v[pl.ds(qo + gi * L, L)]
        x = u * jnp.float32(W - 1)
        y = v * jnp.float32(H - 1)
        x0 = jnp.minimum(x.astype(jnp.int32), W - 1)
        y0 = jnp.minimum(y.astype(jnp.int32), H - 1)
        x1 = jnp.minimum(x0 + 1, W - 1)
        y1 = jnp.minimum(y0 + 1, H - 1)
        wx = x - x0.astype(jnp.float32)
        wy = y - y0.astype(jnp.float32)
        omx = 1.0 - wx
        omy = 1.0 - wy
        xr0 = x0 * H
        xr1 = x1 * H
        i00[s] = xr0 + y0
        i10[s] = xr1 + y0
        i01[s] = xr0 + y1
        i11[s] = xr1 + y1
        w00[s] = omx * omy
        w10[s] = wx * omy
        w01[s] = omx * wy
        w11[s] = wx * wy
        return 0
    lax.fori_loop(0, CHUNK // L, grp, 0, unroll=2)


def _combine(f00, f10, f01, f11, w00, w10, w01, w11, out_v):
    def grp(gi, _):
        s = pl.ds(gi * L, L)
        w00v = w00[s]
        w10v = w10[s]
        w01v = w01[s]
        w11v = w11[s]
        for k in range(L):
            j = gi * L + k
            acc = (f00[j, :] * w00v[k] + f10[j, :] * w10v[k]
                   + f01[j, :] * w01v[k] + f11[j, :] * w11v[k])
            out_v[j, :] = acc
        return 0
    lax.fori_loop(0, CHUNK // L, grp, 0)


def _sampler_body(tex_hbm, u_hbm, v_hbm, out_hbm,
                  tab_hbm,
                  chbA, chbB, rowsA, rowsB,
                  uvA, uvB,
                  iA, iB, wA, wB,
                  fA, fB,
                  outA, outB,
                  sem_iA, sem_iB, sem_oA, sem_oB,
                  sem_uA, sem_uB, sem_gA, sem_gB):
    cc = lax.axis_index("c")
    sid = lax.axis_index("s")
    coff = cc * (W * H)

    io1 = lax.iota(jnp.int32, L)

    def fire_in(bank, sem, x, h):
        return [pltpu.async_copy(
            tex_hbm.at[c16, x, pl.ds(h * HH, HH)], bank.at[c16], sem)
            for c16 in range(C)]

    def drain_in(bank, sem):
        for c16 in range(C):
            pltpu.make_async_copy(tex_hbm.at[0, 0, pl.ds(0, HH)],
                                  bank.at[c16], sem).wait()

    x0r = sid * XPT
    fire_in(chbA, sem_iA, x0r, 0)
    fire_in(chbB, sem_iB, x0r, 1)

    def xrow_body(r, _):
        x = sid * XPT + r

        @pl.when(r > 0)
        def _():
            pltpu.make_async_copy(rowsA, tab_hbm.at[pl.ds(0, HH)],
                                  sem_oA).wait()
            pltpu.make_async_copy(rowsB, tab_hbm.at[pl.ds(0, HH)],
                                  sem_oB).wait()

        drain_in(chbA, sem_iA)
        _interleave(chbA, rowsA, io1)

        @pl.when(r < XPT - 1)
        def _():
            fire_in(chbA, sem_iA, x + 1, 0)

        pltpu.async_copy(rowsA, tab_hbm.at[pl.ds(coff + x * H, HH)], sem_oA)

        drain_in(chbB, sem_iB)
        _interleave(chbB, rowsB, io1)

        @pl.when(r < XPT - 1)
        def _():
            fire_in(chbB, sem_iB, x + 1, 1)

        pltpu.async_copy(rowsB, tab_hbm.at[pl.ds(coff + x * H + HH, HH)],
                         sem_oB)
        return 0

    lax.fori_loop(0, XPT, xrow_body, 0)
    pltpu.make_async_copy(rowsA, tab_hbm.at[pl.ds(0, HH)], sem_oA).wait()
    pltpu.make_async_copy(rowsB, tab_hbm.at[pl.ds(0, HH)], sem_oB).wait()
    plsc.subcore_barrier()

    wid = sid * 2 + cc
    qbase = wid * PER_W
    tabv = tab_hbm

    def fire_uv(bank, sem, c):
        off = qbase + c * CHUNK
        pltpu.async_copy(u_hbm.at[pl.ds(off, CHUNK)], bank.at[0], sem)
        pltpu.async_copy(v_hbm.at[pl.ds(off, CHUNK)], bank.at[1], sem)

    def drain_uv(bank, sem):
        pltpu.make_async_copy(u_hbm.at[pl.ds(0, CHUNK)], bank.at[0],
                              sem).wait()
        pltpu.make_async_copy(u_hbm.at[pl.ds(0, CHUNK)], bank.at[1],
                              sem).wait()

    def fire_gathers(ib, fb, sem):
        for t in range(4):
            pltpu.async_copy(tabv.at[ib.at[t]], fb.at[t], sem)

    def drain_gathers(ib, fb, sem):
        for t in range(4):
            pltpu.make_async_copy(tabv.at[ib.at[t]], fb.at[t], sem).wait()

    def compute(uvb, ib, wb):
        _indices_weights(uvb.at[0], uvb.at[1], 0,
                         ib.at[0], ib.at[1], ib.at[2], ib.at[3],
                         wb.at[0], wb.at[1], wb.at[2], wb.at[3])

    def combine_store(fb, wb, ob, sem, c):
        _combine(fb.at[0], fb.at[1], fb.at[2], fb.at[3],
                 wb.at[0], wb.at[1], wb.at[2], wb.at[3], ob)
        pltpu.async_copy(ob, out_hbm.at[pl.ds(qbase + c * CHUNK, CHUNK)], sem)

    fire_uv(uvA, sem_uA, 0)
    fire_uv(uvB, sem_uB, 1)
    drain_uv(uvA, sem_uA)
    compute(uvA, iA, wA)
    fire_gathers(iA, fA, sem_gA)

    def chunk_body(i, _):
        c0 = 2 * i
        drain_uv(uvB, sem_uB)
        compute(uvB, iB, wB)
        fire_gathers(iB, fB, sem_gB)
        @pl.when(i > 0)
        def _():
            pltpu.make_async_copy(outA, out_hbm.at[pl.ds(0, CHUNK)],
                                  sem_oA).wait()
        drain_gathers(iA, fA, sem_gA)
        combine_store(fA, wA, outA, sem_oA, c0)
        @pl.when(i < NCHUNK // 2 - 1)
        def _():
            fire_uv(uvA, sem_uA, c0 + 2)
            fire_uv(uvB, sem_uB, c0 + 3)
        @pl.when(i > 0)
        def _():
            pltpu.make_async_copy(outB, out_hbm.at[pl.ds(0, CHUNK)],
                                  sem_oB).wait()
        drain_gathers(iB, fB, sem_gB)
        combine_store(fB, wB, outB, sem_oB, c0 + 1)
        @pl.when(i < NCHUNK // 2 - 1)
        def _():
            drain_uv(uvA, sem_uA)
            compute(uvA, iA, wA)
            fire_gathers(iA, fA, sem_gA)
        return 0

    lax.fori_loop(0, NCHUNK // 2, chunk_body, 0)
    pltpu.make_async_copy(outA, out_hbm.at[pl.ds(0, CHUNK)], sem_oA).wait()
    pltpu.make_async_copy(outB, out_hbm.at[pl.ds(0, CHUNK)], sem_oB).wait()


def kernel(input, param):
    u = param[:, 0]
    v = param[:, 1]

    mesh = plsc.VectorSubcoreMesh(core_axis_name="c", subcore_axis_name="s")
    f = pl.kernel(
        _sampler_body,
        out_type=jax.ShapeDtypeStruct((N, C), jnp.float32),
        mesh=mesh,
        compiler_params=pltpu.CompilerParams(use_tc_tiling_on_sc=False,
                                             needs_layout_passes=False),
        scratch_types=[
            pltpu.HBM((2 * W * H, C), jnp.float32),
            pltpu.VMEM((C, HH), jnp.float32),
            pltpu.VMEM((C, HH), jnp.float32),
            pltpu.VMEM((HH, C), jnp.float32),
            pltpu.VMEM((HH, C), jnp.float32),
            pltpu.VMEM((2, CHUNK), jnp.float32),
            pltpu.VMEM((2, CHUNK), jnp.float32),
            pltpu.VMEM((4, CHUNK), jnp.int32),
            pltpu.VMEM((4, CHUNK), jnp.int32),
            pltpu.VMEM((4, CHUNK), jnp.float32),
            pltpu.VMEM((4, CHUNK), jnp.float32),
            pltpu.VMEM((4, CHUNK, C), jnp.float32),
            pltpu.VMEM((4, CHUNK, C), jnp.float32),
            pltpu.VMEM((CHUNK, C), jnp.float32),
            pltpu.VMEM((CHUNK, C), jnp.float32),
            pltpu.SemaphoreType.DMA,
            pltpu.SemaphoreType.DMA,
            pltpu.SemaphoreType.DMA,
            pltpu.SemaphoreType.DMA,
            pltpu.SemaphoreType.DMA,
            pltpu.SemaphoreType.DMA,
            pltpu.SemaphoreType.DMA,
            pltpu.SemaphoreType.DMA,
        ],
    )
    return f(input, u, v)


# speedup vs baseline: 1.2506x; 1.0009x over previous
import functools

import jax
import jax.numpy as jnp
from jax import lax
from jax.experimental import pallas as pl
from jax.experimental.pallas import tpu as pltpu
from jax.experimental.pallas import tpu_sc as plsc

C = 16
W = 2048
H = 2048
N = 1048576
NW = 32
PER_W = N // NW
CHUNK = 256
NCHUNK = PER_W // CHUNK
L = 16
XPT = W // 16
HH = 1024


def _interleave(chb, rows, io1):
    def grp(g, _):
        rowi = io1 + g * L
        for c16 in range(C):
            v = chb[c16, pl.ds(g * L, L)]
            plsc.store_scatter(
                rows, [rowi, jnp.full((L,), c16, jnp.int32)], v)
        return 0
    lax.fori_loop(0, HH // L, grp, 0)


def _indices_weights(u_v, v_v, qo, i00, i10, i01, i11, w00, w10, w01, w11):
    def grp(gi, _):
        s = pl.ds(gi * L, L)
        u = u_v[pl.ds(qo + gi * L, L)]
        v = v_v[pl.ds(qo + gi * L, L)]
        x = u * jnp.float32(W - 1)
        y = v * jnp.float32(H - 1)
        x0 = jnp.minimum(x.astype(jnp.int32), W - 1)
        y0 = jnp.minimum(y.astype(jnp.int32), H - 1)
        x1 = jnp.minimum(x0 + 1, W - 1)
        y1 = jnp.minimum(y0 + 1, H - 1)
        wx = x - x0.astype(jnp.float32)
        wy = y - y0.astype(jnp.float32)
        omx = 1.0 - wx
        omy = 1.0 - wy
        xr0 = x0 * H
        xr1 = x1 * H
        i00[s] = xr0 + y0
        i10[s] = xr1 + y0
        i01[s] = xr0 + y1
        i11[s] = xr1 + y1
        w00[s] = omx * omy
        w10[s] = wx * omy
        w01[s] = omx * wy
        w11[s] = wx * wy
        return 0
    lax.fori_loop(0, CHUNK // L, grp, 0, unroll=2)


def _combine(f00, f10, f01, f11, w00, w10, w01, w11, out_v):
    def grp(gi, _):
        s = pl.ds(gi * L, L)
        w00v = w00[s]
        w10v = w10[s]
        w01v = w01[s]
        w11v = w11[s]
        for k in range(L):
            j = gi * L + k
            acc = (f00[j, :] * w00v[k] + f10[j, :] * w10v[k]
                   + f01[j, :] * w01v[k] + f11[j, :] * w11v[k])
            out_v[gi * 2 + k // 8, pl.ds((k % 8) * C, C)] = acc
        return 0
    lax.fori_loop(0, CHUNK // L, grp, 0)


def _sampler_body(tex_hbm, u_hbm, v_hbm, out_hbm,
                  tab_hbm,
                  chbA, chbB, rowsA, rowsB,
                  uvA, uvB,
                  iA, iB, wA, wB,
                  fA, fB,
                  outA, outB,
                  sem_iA, sem_iB, sem_oA, sem_oB,
                  sem_uA, sem_uB, sem_gA, sem_gB):
    cc = lax.axis_index("c")
    sid = lax.axis_index("s")
    coff = cc * (W * H)

    io1 = lax.iota(jnp.int32, L)

    def fire_in(bank, sem, x, h):
        return [pltpu.async_copy(
            tex_hbm.at[c16, x, pl.ds(h * HH, HH)], bank.at[c16], sem)
            for c16 in range(C)]

    def drain_in(bank, sem):
        for c16 in range(C):
            pltpu.make_async_copy(tex_hbm.at[0, 0, pl.ds(0, HH)],
                                  bank.at[c16], sem).wait()

    x0r = sid * XPT
    fire_in(chbA, sem_iA, x0r, 0)
    fire_in(chbB, sem_iB, x0r, 1)

    def xrow_body(r, _):
        x = sid * XPT + r

        @pl.when(r > 0)
        def _():
            pltpu.make_async_copy(rowsA, tab_hbm.at[pl.ds(0, HH)],
                                  sem_oA).wait()
            pltpu.make_async_copy(rowsB, tab_hbm.at[pl.ds(0, HH)],
                                  sem_oB).wait()

        drain_in(chbA, sem_iA)
        _interleave(chbA, rowsA, io1)

        @pl.when(r < XPT - 1)
        def _():
            fire_in(chbA, sem_iA, x + 1, 0)

        pltpu.async_copy(rowsA, tab_hbm.at[pl.ds(coff + x * H, HH)], sem_oA)

        drain_in(chbB, sem_iB)
        _interleave(chbB, rowsB, io1)

        @pl.when(r < XPT - 1)
        def _():
            fire_in(chbB, sem_iB, x + 1, 1)

        pltpu.async_copy(rowsB, tab_hbm.at[pl.ds(coff + x * H + HH, HH)],
                         sem_oB)
        return 0

    lax.fori_loop(0, XPT, xrow_body, 0)
    pltpu.make_async_copy(rowsA, tab_hbm.at[pl.ds(0, HH)], sem_oA).wait()
    pltpu.make_async_copy(rowsB, tab_hbm.at[pl.ds(0, HH)], sem_oB).wait()
    plsc.subcore_barrier()

    wid = sid * 2 + cc
    qbase = wid * PER_W
    tabv = tab_hbm

    def fire_uv(bank, sem, c):
        off = qbase + c * CHUNK
        pltpu.async_copy(u_hbm.at[pl.ds(off, CHUNK)], bank.at[0], sem)
        pltpu.async_copy(v_hbm.at[pl.ds(off, CHUNK)], bank.at[1], sem)

    def drain_uv(bank, sem):
        pltpu.make_async_copy(u_hbm.at[pl.ds(0, CHUNK)], bank.at[0],
                              sem).wait()
        pltpu.make_async_copy(u_hbm.at[pl.ds(0, CHUNK)], bank.at[1],
                              sem).wait()

    def fire_gathers(ib, fb, sem):
        for t in range(4):
            pltpu.async_copy(tabv.at[ib.at[t]], fb.at[t], sem)

    def drain_gathers(ib, fb, sem):
        for t in range(4):
            pltpu.make_async_copy(tabv.at[ib.at[t]], fb.at[t], sem).wait()

    def compute(uvb, ib, wb):
        _indices_weights(uvb.at[0], uvb.at[1], 0,
                         ib.at[0], ib.at[1], ib.at[2], ib.at[3],
                         wb.at[0], wb.at[1], wb.at[2], wb.at[3])

    def combine_store(fb, wb, ob, sem, c):
        _combine(fb.at[0], fb.at[1], fb.at[2], fb.at[3],
                 wb.at[0], wb.at[1], wb.at[2], wb.at[3], ob)
        pltpu.async_copy(
            ob, out_hbm.at[pl.ds((qbase + c * CHUNK) // 8, CHUNK // 8)], sem)

    fire_uv(uvA, sem_uA, 0)
    fire_uv(uvB, sem_uB, 1)
    drain_uv(uvA, sem_uA)
    compute(uvA, iA, wA)
    fire_gathers(iA, fA, sem_gA)

    def chunk_body(i, _):
        c0 = 2 * i
        drain_uv(uvB, sem_uB)
        compute(uvB, iB, wB)
        fire_gathers(iB, fB, sem_gB)
        @pl.when(i > 0)
        def _():
            pltpu.make_async_copy(outA, out_hbm.at[pl.ds(0, CHUNK // 8)],
                                  sem_oA).wait()
        drain_gathers(iA, fA, sem_gA)
        combine_store(fA, wA, outA, sem_oA, c0)
        @pl.when(i < NCHUNK // 2 - 1)
        def _():
            fire_uv(uvA, sem_uA, c0 + 2)
            fire_uv(uvB, sem_uB, c0 + 3)
        @pl.when(i > 0)
        def _():
            pltpu.make_async_copy(outB, out_hbm.at[pl.ds(0, CHUNK // 8)],
                                  sem_oB).wait()
        drain_gathers(iB, fB, sem_gB)
        combine_store(fB, wB, outB, sem_oB, c0 + 1)
        @pl.when(i < NCHUNK // 2 - 1)
        def _():
            drain_uv(uvA, sem_uA)
            compute(uvA, iA, wA)
            fire_gathers(iA, fA, sem_gA)
        return 0

    lax.fori_loop(0, NCHUNK // 2, chunk_body, 0)
    pltpu.make_async_copy(outA, out_hbm.at[pl.ds(0, CHUNK // 8)], sem_oA).wait()
    pltpu.make_async_copy(outB, out_hbm.at[pl.ds(0, CHUNK // 8)], sem_oB).wait()


def kernel(input, param):
    u = param[:, 0]
    v = param[:, 1]

    mesh = plsc.VectorSubcoreMesh(core_axis_name="c", subcore_axis_name="s")
    f = pl.kernel(
        _sampler_body,
        out_type=jax.ShapeDtypeStruct((N * C // 128, 128), jnp.float32),
        mesh=mesh,
        compiler_params=pltpu.CompilerParams(use_tc_tiling_on_sc=False,
                                             needs_layout_passes=False),
        scratch_types=[
            pltpu.HBM((2 * W * H, C), jnp.float32),
            pltpu.VMEM((C, HH), jnp.float32),
            pltpu.VMEM((C, HH), jnp.float32),
            pltpu.VMEM((HH, C), jnp.float32),
            pltpu.VMEM((HH, C), jnp.float32),
            pltpu.VMEM((2, CHUNK), jnp.float32),
            pltpu.VMEM((2, CHUNK), jnp.float32),
            pltpu.VMEM((4, CHUNK), jnp.int32),
            pltpu.VMEM((4, CHUNK), jnp.int32),
            pltpu.VMEM((4, CHUNK), jnp.float32),
            pltpu.VMEM((4, CHUNK), jnp.float32),
            pltpu.VMEM((4, CHUNK, C), jnp.float32),
            pltpu.VMEM((4, CHUNK, C), jnp.float32),
            pltpu.VMEM((CHUNK // 8, 128), jnp.float32),
            pltpu.VMEM((CHUNK // 8, 128), jnp.float32),
            pltpu.SemaphoreType.DMA,
            pltpu.SemaphoreType.DMA,
            pltpu.SemaphoreType.DMA,
            pltpu.SemaphoreType.DMA,
            pltpu.SemaphoreType.DMA,
            pltpu.SemaphoreType.DMA,
            pltpu.SemaphoreType.DMA,
            pltpu.SemaphoreType.DMA,
        ],
    )
    return f(input, u, v).reshape(N, C)


# speedup vs baseline: 1.3023x; 1.0413x over previous
import functools

import jax
import jax.numpy as jnp
from jax import lax
from jax.experimental import pallas as pl
from jax.experimental.pallas import tpu as pltpu
from jax.experimental.pallas import tpu_sc as plsc

C = 16
W = 2048
H = 2048
N = 1048576
NW = 32
PER_W = N // NW
CHUNK = 256
NCHUNK = PER_W // CHUNK
L = 16
XPT = W // 16
HH = 1024


def _interleave(chb, rows, io1):
    def grp(g, _):
        rowi = io1 + g * L
        for c16 in range(C):
            v = chb[c16, pl.ds(g * L, L)]
            plsc.store_scatter(
                rows, [rowi, jnp.full((L,), c16, jnp.int32)], v)
        return 0
    lax.fori_loop(0, HH // L, grp, 0)


def _indices_weights(u_v, v_v, qo, i00, i10, i01, i11, w00, w10, w01, w11):
    def grp(gi, _):
        s = pl.ds(gi * L, L)
        u = u_v[pl.ds(qo + gi * L, L)]
        v = v_v[pl.ds(qo + gi * L, L)]
        x = u * jnp.float32(W - 1)
        y = v * jnp.float32(H - 1)
        x0 = jnp.minimum(x.astype(jnp.int32), W - 1)
        y0 = jnp.minimum(y.astype(jnp.int32), H - 1)
        x1 = jnp.minimum(x0 + 1, W - 1)
        y1 = jnp.minimum(y0 + 1, H - 1)
        wx = x - x0.astype(jnp.float32)
        wy = y - y0.astype(jnp.float32)
        omx = 1.0 - wx
        omy = 1.0 - wy
        xr0 = x0 * H
        xr1 = x1 * H
        i00[s] = xr0 + y0
        i10[s] = xr1 + y0
        i01[s] = xr0 + y1
        i11[s] = xr1 + y1
        w00[s] = omx * omy
        w10[s] = wx * omy
        w01[s] = omx * wy
        w11[s] = wx * wy
        return 0
    lax.fori_loop(0, CHUNK // L, grp, 0, unroll=2)


def _combine(f00, f10, f01, f11, w00, w10, w01, w11, out_v):
    def grp(gi, _):
        s = pl.ds(gi * L, L)
        w00v = w00[s]
        w10v = w10[s]
        w01v = w01[s]
        w11v = w11[s]
        for k in range(L):
            j = gi * L + k
            acc = (f00[j, :] * w00v[k] + f10[j, :] * w10v[k]
                   + f01[j, :] * w01v[k] + f11[j, :] * w11v[k])
            out_v[gi * 2 + k // 8, pl.ds((k % 8) * C, C)] = acc
        return 0
    lax.fori_loop(0, CHUNK // L, grp, 0)


def _sampler_body(tex_hbm, u_hbm, v_hbm, out_hbm,
                  tab_hbm,
                  chbA, chbB, rowsA, rowsB,
                  uvA, uvB,
                  iA, iB, wA, wB,
                  fA, fB,
                  outA, outB,
                  sem_iA, sem_iB, sem_oA, sem_oB,
                  sem_uA, sem_uB, sem_gA, sem_gB):
    cc = lax.axis_index("c")
    sid = lax.axis_index("s")
    coff = cc * (W * H)

    io1 = lax.iota(jnp.int32, L)

    def fire_in(bank, sem, x, h):
        return [pltpu.async_copy(
            tex_hbm.at[c16, x, pl.ds(h * HH, HH)], bank.at[c16], sem)
            for c16 in range(C)]

    def drain_in(bank, sem):
        for c16 in range(C):
            pltpu.make_async_copy(tex_hbm.at[0, 0, pl.ds(0, HH)],
                                  bank.at[c16], sem).wait()

    x0r = sid * XPT
    fire_in(chbA, sem_iA, x0r, 0)
    fire_in(chbB, sem_iB, x0r, 1)

    def xrow_body(r, _):
        x = sid * XPT + r

        @pl.when(r > 0)
        def _():
            pltpu.make_async_copy(rowsA, tab_hbm.at[pl.ds(0, HH)],
                                  sem_oA).wait()
            pltpu.make_async_copy(rowsB, tab_hbm.at[pl.ds(0, HH)],
                                  sem_oB).wait()

        drain_in(chbA, sem_iA)
        _interleave(chbA, rowsA, io1)

        @pl.when(r < XPT - 1)
        def _():
            fire_in(chbA, sem_iA, x + 1, 0)

        pltpu.async_copy(rowsA, tab_hbm.at[pl.ds(coff + x * H, HH)], sem_oA)

        drain_in(chbB, sem_iB)
        _interleave(chbB, rowsB, io1)

        @pl.when(r < XPT - 1)
        def _():
            fire_in(chbB, sem_iB, x + 1, 1)

        pltpu.async_copy(rowsB, tab_hbm.at[pl.ds(coff + x * H + HH, HH)],
                         sem_oB)
        return 0

    lax.fori_loop(0, XPT, xrow_body, 0)
    pltpu.make_async_copy(rowsA, tab_hbm.at[pl.ds(0, HH)], sem_oA).wait()
    pltpu.make_async_copy(rowsB, tab_hbm.at[pl.ds(0, HH)], sem_oB).wait()
    plsc.subcore_barrier()

    wid = sid * 2 + cc
    qbase = wid * PER_W
    tabv = tab_hbm

    def fire_uv(bank, sem, c):
        off = qbase + c * CHUNK
        pltpu.async_copy(u_hbm.at[pl.ds(off, CHUNK)], bank.at[0], sem)
        pltpu.async_copy(v_hbm.at[pl.ds(off, CHUNK)], bank.at[1], sem)

    def drain_uv(bank, sem):
        pltpu.make_async_copy(u_hbm.at[pl.ds(0, CHUNK)], bank.at[0],
                              sem).wait()
        pltpu.make_async_copy(u_hbm.at[pl.ds(0, CHUNK)], bank.at[1],
                              sem).wait()

    def fire_gathers(ib, fb, sem):
        for t in range(4):
            pltpu.async_copy(tabv.at[ib.at[t]], fb.at[t], sem)

    def drain_gathers(ib, fb, sem):
        for t in range(4):
            pltpu.make_async_copy(tabv.at[ib.at[t]], fb.at[t], sem).wait()

    def compute(uvb, ib, wb):
        _indices_weights(uvb.at[0], uvb.at[1], 0,
                         ib.at[0], ib.at[1], ib.at[2], ib.at[3],
                         wb.at[0], wb.at[1], wb.at[2], wb.at[3])

    def combine_store(fb, wb, ob, sem, c):
        _combine(fb.at[0], fb.at[1], fb.at[2], fb.at[3],
                 wb.at[0], wb.at[1], wb.at[2], wb.at[3], ob)
        pltpu.async_copy(
            ob, out_hbm.at[pl.ds((qbase + c * CHUNK) // 8, CHUNK // 8)], sem)

    fire_uv(uvA, sem_uA, 0)
    fire_uv(uvB, sem_uB, 1)
    drain_uv(uvA, sem_uA)
    compute(uvA, iA, wA)
    fire_gathers(iA, fA, sem_gA)

    def chunk_body(i, _):
        c0 = 2 * i
        drain_uv(uvB, sem_uB)
        compute(uvB, iB, wB)
        fire_gathers(iB, fB, sem_gB)
        @pl.when(i > 0)
        def _():
            pltpu.make_async_copy(outA, out_hbm.at[pl.ds(0, CHUNK // 8)],
                                  sem_oA).wait()
        drain_gathers(iA, fA, sem_gA)
        combine_store(fA, wA, outA, sem_oA, c0)
        @pl.when(i < NCHUNK // 2 - 1)
        def _():
            fire_uv(uvA, sem_uA, c0 + 2)
            fire_uv(uvB, sem_uB, c0 + 3)
        @pl.when(i > 0)
        def _():
            pltpu.make_async_copy(outB, out_hbm.at[pl.ds(0, CHUNK // 8)],
                                  sem_oB).wait()
        drain_gathers(iB, fB, sem_gB)
        combine_store(fB, wB, outB, sem_oB, c0 + 1)
        @pl.when(i < NCHUNK // 2 - 1)
        def _():
            drain_uv(uvA, sem_uA)
            compute(uvA, iA, wA)
            fire_gathers(iA, fA, sem_gA)
        return 0

    lax.fori_loop(0, NCHUNK // 2, chunk_body, 0)
    pltpu.make_async_copy(outA, out_hbm.at[pl.ds(0, CHUNK // 8)], sem_oA).wait()
    pltpu.make_async_copy(outB, out_hbm.at[pl.ds(0, CHUNK // 8)], sem_oB).wait()


def kernel(input, param):
    u = param[:, 0]
    v = param[:, 1]

    mesh = plsc.VectorSubcoreMesh(core_axis_name="c", subcore_axis_name="s")
    f = pl.kernel(
        _sampler_body,
        out_type=jax.ShapeDtypeStruct((N * C // 128, 128), jnp.float32),
        mesh=mesh,
        compiler_params=pltpu.CompilerParams(use_tc_tiling_on_sc=False,
                                             needs_layout_passes=False),
        scratch_types=[
            pltpu.HBM((2 * W * H, C), jnp.float32),
            pltpu.VMEM((C, HH), jnp.float32),
            pltpu.VMEM((C, HH), jnp.float32),
            pltpu.VMEM((HH, C), jnp.float32),
            pltpu.VMEM((HH, C), jnp.float32),
            pltpu.VMEM((2, CHUNK), jnp.float32),
            pltpu.VMEM((2, CHUNK), jnp.float32),
            pltpu.VMEM((4, CHUNK), jnp.int32),
            pltpu.VMEM((4, CHUNK), jnp.int32),
            pltpu.VMEM((4, CHUNK), jnp.float32),
            pltpu.VMEM((4, CHUNK), jnp.float32),
            pltpu.VMEM((4, CHUNK, C), jnp.float32),
            pltpu.VMEM((4, CHUNK, C), jnp.float32),
            pltpu.VMEM((CHUNK // 8, 128), jnp.float32),
            pltpu.VMEM((CHUNK // 8, 128), jnp.float32),
            pltpu.SemaphoreType.DMA,
            pltpu.SemaphoreType.DMA,
            pltpu.SemaphoreType.DMA,
            pltpu.SemaphoreType.DMA,
            pltpu.SemaphoreType.DMA,
            pltpu.SemaphoreType.DMA,
            pltpu.SemaphoreType.DMA,
            pltpu.SemaphoreType.DMA,
        ],
    )
    out = f(input, u, v).reshape(N, C)
    return out @ jnp.eye(C, dtype=jnp.float32)


# speedup vs baseline: 1.5591x; 1.1972x over previous
import functools

import jax
import jax.numpy as jnp
from jax import lax
from jax.experimental import pallas as pl
from jax.experimental.pallas import tpu as pltpu
from jax.experimental.pallas import tpu_sc as plsc

C = 16
W = 2048
H = 2048
N = 1048576
NW = 32
PER_W = N // NW
CHUNK = 256
NCHUNK = PER_W // CHUNK
L = 16
XPT = W // NW
HH = 1024


def _interleave(chb, rows, io1):
    def grp(g, _):
        rowi = io1 + g * L
        for c16 in range(C):
            v = chb[c16, pl.ds(g * L, L)]
            plsc.store_scatter(
                rows, [rowi, jnp.full((L,), c16, jnp.int32)], v)
        return 0
    lax.fori_loop(0, HH // L, grp, 0)


def _indices_weights(u_v, v_v, qo, i00, i10, i01, i11, w00, w10, w01, w11):
    def grp(gi, _):
        s = pl.ds(gi * L, L)
        u = u_v[pl.ds(qo + gi * L, L)]
        v = v_v[pl.ds(qo + gi * L, L)]
        x = u * jnp.float32(W - 1)
        y = v * jnp.float32(H - 1)
        x0 = jnp.minimum(x.astype(jnp.int32), W - 1)
        y0 = jnp.minimum(y.astype(jnp.int32), H - 1)
        x1 = jnp.minimum(x0 + 1, W - 1)
        y1 = jnp.minimum(y0 + 1, H - 1)
        wx = x - x0.astype(jnp.float32)
        wy = y - y0.astype(jnp.float32)
        omx = 1.0 - wx
        omy = 1.0 - wy
        xr0 = x0 * H
        xr1 = x1 * H
        i00[s] = xr0 + y0
        i10[s] = xr1 + y0
        i01[s] = xr0 + y1
        i11[s] = xr1 + y1
        w00[s] = omx * omy
        w10[s] = wx * omy
        w01[s] = omx * wy
        w11[s] = wx * wy
        return 0
    lax.fori_loop(0, CHUNK // L, grp, 0, unroll=2)


def _combine(f00, f10, f01, f11, w00, w10, w01, w11, out_v):
    def grp(gi, _):
        s = pl.ds(gi * L, L)
        w00v = w00[s]
        w10v = w10[s]
        w01v = w01[s]
        w11v = w11[s]
        for k in range(L):
            j = gi * L + k
            acc = (f00[j, :] * w00v[k] + f10[j, :] * w10v[k]
                   + f01[j, :] * w01v[k] + f11[j, :] * w11v[k])
            out_v[gi * 2 + k // 8, pl.ds((k % 8) * C, C)] = acc
        return 0
    lax.fori_loop(0, CHUNK // L, grp, 0)


def _sampler_body(tex_hbm, u_hbm, v_hbm, out_hbm,
                  tab_hbm,
                  chbA, chbB, rowsA, rowsB,
                  uvA, uvB,
                  iA, iB, wA, wB,
                  fA, fB,
                  outA, outB,
                  sem_iA, sem_iB, sem_oA, sem_oB,
                  sem_uA, sem_uB, sem_gA, sem_gB, sem_x):
    cc = lax.axis_index("c")
    sid = lax.axis_index("s")
    wid = sid * 2 + cc

    io1 = lax.iota(jnp.int32, L)

    def fire_in(bank, sem, x, h):
        return [pltpu.async_copy(
            tex_hbm.at[c16, x, pl.ds(h * HH, HH)], bank.at[c16], sem)
            for c16 in range(C)]

    def drain_in(bank, sem):
        for c16 in range(C):
            pltpu.make_async_copy(tex_hbm.at[0, 0, pl.ds(0, HH)],
                                  bank.at[c16], sem).wait()

    x0r = wid * XPT
    fire_in(chbA, sem_iA, x0r, 0)
    fire_in(chbB, sem_iB, x0r, 1)

    def xrow_body(r, _):
        x = wid * XPT + r

        @pl.when(r > 0)
        def _():
            pltpu.make_async_copy(rowsA, tab_hbm.at[pl.ds(0, HH)],
                                  sem_oA).wait()
            pltpu.make_async_copy(rowsB, tab_hbm.at[pl.ds(0, HH)],
                                  sem_oB).wait()

        drain_in(chbA, sem_iA)
        _interleave(chbA, rowsA, io1)

        @pl.when(r < XPT - 1)
        def _():
            fire_in(chbA, sem_iA, x + 1, 0)

        pltpu.async_copy(rowsA, tab_hbm.at[pl.ds(x * H, HH)], sem_oA)

        drain_in(chbB, sem_iB)
        _interleave(chbB, rowsB, io1)

        @pl.when(r < XPT - 1)
        def _():
            fire_in(chbB, sem_iB, x + 1, 1)

        pltpu.async_copy(rowsB, tab_hbm.at[pl.ds(x * H + HH, HH)], sem_oB)
        return 0

    lax.fori_loop(0, XPT, xrow_body, 0)
    pltpu.make_async_copy(rowsA, tab_hbm.at[pl.ds(0, HH)], sem_oA).wait()
    pltpu.make_async_copy(rowsB, tab_hbm.at[pl.ds(0, HH)], sem_oB).wait()
    plsc.subcore_barrier()
    pltpu.semaphore_signal(sem_x, 1, device_id={"c": 1 - cc},
                           device_id_type=pl.DeviceIdType.MESH)
    pltpu.semaphore_wait(sem_x, 1)

    qbase = wid * PER_W
    tabv = tab_hbm

    def fire_uv(bank, sem, c):
        off = qbase + c * CHUNK
        pltpu.async_copy(u_hbm.at[pl.ds(off, CHUNK)], bank.at[0], sem)
        pltpu.async_copy(v_hbm.at[pl.ds(off, CHUNK)], bank.at[1], sem)

    def drain_uv(bank, sem):
        pltpu.make_async_copy(u_hbm.at[pl.ds(0, CHUNK)], bank.at[0],
                              sem).wait()
        pltpu.make_async_copy(u_hbm.at[pl.ds(0, CHUNK)], bank.at[1],
                              sem).wait()

    def fire_gathers(ib, fb, sem):
        for t in range(4):
            pltpu.async_copy(tabv.at[ib.at[t]], fb.at[t], sem)

    def drain_gathers(ib, fb, sem):
        for t in range(4):
            pltpu.make_async_copy(tabv.at[ib.at[t]], fb.at[t], sem).wait()

    def compute(uvb, ib, wb):
        _indices_weights(uvb.at[0], uvb.at[1], 0,
                         ib.at[0], ib.at[1], ib.at[2], ib.at[3],
                         wb.at[0], wb.at[1], wb.at[2], wb.at[3])

    def combine_store(fb, wb, ob, sem, c):
        _combine(fb.at[0], fb.at[1], fb.at[2], fb.at[3],
                 wb.at[0], wb.at[1], wb.at[2], wb.at[3], ob)
        pltpu.async_copy(
            ob, out_hbm.at[pl.ds((qbase + c * CHUNK) // 8, CHUNK // 8)], sem)

    fire_uv(uvA, sem_uA, 0)
    fire_uv(uvB, sem_uB, 1)
    drain_uv(uvA, sem_uA)
    compute(uvA, iA, wA)
    fire_gathers(iA, fA, sem_gA)

    def chunk_body(i, _):
        c0 = 2 * i
        drain_uv(uvB, sem_uB)
        compute(uvB, iB, wB)
        fire_gathers(iB, fB, sem_gB)
        @pl.when(i > 0)
        def _():
            pltpu.make_async_copy(outA, out_hbm.at[pl.ds(0, CHUNK // 8)],
                                  sem_oA).wait()
        drain_gathers(iA, fA, sem_gA)
        combine_store(fA, wA, outA, sem_oA, c0)
        @pl.when(i < NCHUNK // 2 - 1)
        def _():
            fire_uv(uvA, sem_uA, c0 + 2)
            fire_uv(uvB, sem_uB, c0 + 3)
        @pl.when(i > 0)
        def _():
            pltpu.make_async_copy(outB, out_hbm.at[pl.ds(0, CHUNK // 8)],
                                  sem_oB).wait()
        drain_gathers(iB, fB, sem_gB)
        combine_store(fB, wB, outB, sem_oB, c0 + 1)
        @pl.when(i < NCHUNK // 2 - 1)
        def _():
            drain_uv(uvA, sem_uA)
            compute(uvA, iA, wA)
            fire_gathers(iA, fA, sem_gA)
        return 0

    lax.fori_loop(0, NCHUNK // 2, chunk_body, 0)
    pltpu.make_async_copy(outA, out_hbm.at[pl.ds(0, CHUNK // 8)], sem_oA).wait()
    pltpu.make_async_copy(outB, out_hbm.at[pl.ds(0, CHUNK // 8)], sem_oB).wait()


def kernel(input, param):
    u = param[:, 0]
    v = param[:, 1]

    mesh = plsc.VectorSubcoreMesh(core_axis_name="c", subcore_axis_name="s")
    f = pl.kernel(
        _sampler_body,
        out_type=jax.ShapeDtypeStruct((N * C // 128, 128), jnp.float32),
        mesh=mesh,
        compiler_params=pltpu.CompilerParams(use_tc_tiling_on_sc=False,
                                             needs_layout_passes=False),
        scratch_types=[
            pltpu.HBM((W * H, C), jnp.float32),
            pltpu.VMEM((C, HH), jnp.float32),
            pltpu.VMEM((C, HH), jnp.float32),
            pltpu.VMEM((HH, C), jnp.float32),
            pltpu.VMEM((HH, C), jnp.float32),
            pltpu.VMEM((2, CHUNK), jnp.float32),
            pltpu.VMEM((2, CHUNK), jnp.float32),
            pltpu.VMEM((4, CHUNK), jnp.int32),
            pltpu.VMEM((4, CHUNK), jnp.int32),
            pltpu.VMEM((4, CHUNK), jnp.float32),
            pltpu.VMEM((4, CHUNK), jnp.float32),
            pltpu.VMEM((4, CHUNK, C), jnp.float32),
            pltpu.VMEM((4, CHUNK, C), jnp.float32),
            pltpu.VMEM((CHUNK // 8, 128), jnp.float32),
            pltpu.VMEM((CHUNK // 8, 128), jnp.float32),
            pltpu.SemaphoreType.DMA,
            pltpu.SemaphoreType.DMA,
            pltpu.SemaphoreType.DMA,
            pltpu.SemaphoreType.DMA,
            pltpu.SemaphoreType.DMA,
            pltpu.SemaphoreType.DMA,
            pltpu.SemaphoreType.DMA,
            pltpu.SemaphoreType.DMA,
            pltpu.SemaphoreType.REGULAR,
        ],
    )
    out = f(input, u, v).reshape(N, C)
    return jnp.dot(out, jnp.eye(C, dtype=jnp.float32),
                   precision=jax.lax.Precision.HIGHEST)
